# baseline (speedup 1.0000x reference)
import dataclasses
import functools

import jax
import jax.numpy as jnp
import numpy as np
from jax import lax
from jax.experimental import pallas as pl
from jax.experimental.pallas import tpu as pltpu
from jax.experimental.pallas import tpu_sc as plsc

N = 10000
E = 320000
D = 128
N_CORES = 2
N_SUBCORES = 16
NW = N_CORES * N_SUBCORES
CHUNK = 128
NCH = 80
EPAD = NW * NCH * CHUNK
NPAD = 10240
RPT = NPAD // N_SUBCORES
DROWS = NPAD // D

_mesh = plsc.VectorSubcoreMesh(core_axis_name="c", subcore_axis_name="s")

_cp = pltpu.CompilerParams()
if "needs_layout_passes" in pltpu.CompilerParams.__dataclass_fields__:
    _cp = dataclasses.replace(_cp, needs_layout_passes=False)


@functools.partial(
    pl.kernel,
    out_type=jax.ShapeDtypeStruct((N_CORES, NPAD, D), jnp.float32),
    mesh=_mesh,
    scratch_types=[
        pltpu.VMEM_SHARED((NPAD, D), jnp.float32),
        pltpu.VMEM((8, CHUNK), jnp.int32),
        pltpu.VMEM((8, CHUNK), jnp.int32),
        pltpu.VMEM((CHUNK, D), jnp.float32),
        pltpu.VMEM((CHUNK, D), jnp.float32),
        pltpu.SemaphoreType.DMA,
        pltpu.SemaphoreType.DMA,
        pltpu.SemaphoreType.DMA,
        pltpu.SemaphoreType.DMA,
        pltpu.SemaphoreType.DMA,
        pltpu.SemaphoreType.DMA,
        pltpu.SemaphoreType.DMA,
        pltpu.SemaphoreType.DMA,
        pltpu.SemaphoreType.DMA,
        pltpu.SemaphoreType.DMA,
        pltpu.SemaphoreType.DMA,
        pltpu.SemaphoreType.DMA,
    ],
)
def _hop(x_hbm, src_hbm, dst_hbm, zeros_hbm, part_hbm,
         acc, sidx, didx, rows0, rows1,
         isem0, isem1, isem2, isem3, isem4, isem5, isem6, isem7,
         gsem0, gsem1, ssem0, ssem1):
    cid = lax.axis_index("c")
    sid = lax.axis_index("s")
    wid = cid * N_SUBCORES + sid
    my_rows = pl.ds(sid * RPT, RPT)
    isem = (isem0, isem1, isem2, isem3, isem4, isem5, isem6, isem7)
    rows = (rows0, rows1)
    gsem = (gsem0, gsem1)
    ssem = (ssem0, ssem1)

    def idx_start(g, q):
        pltpu.async_copy(src_hbm.at[wid, g], sidx.at[q], isem[q])
        pltpu.async_copy(dst_hbm.at[wid, g], didx.at[q], isem[q])

    def idx_wait(g, q):
        pltpu.make_async_copy(src_hbm.at[wid, g], sidx.at[q], isem[q]).wait()
        pltpu.make_async_copy(dst_hbm.at[wid, g], didx.at[q], isem[q]).wait()

    def gather_start(q, b):
        pltpu.async_copy(x_hbm.at[sidx.at[q]], rows[b], gsem[b])

    def gather_wait(q, b):
        pltpu.make_async_copy(x_hbm.at[sidx.at[q]], rows[b], gsem[b]).wait()

    def scatter_start(q, b):
        pltpu.async_copy(rows[b], acc.at[didx.at[q]], ssem[b], add=True)

    def scatter_wait(q, b):
        pltpu.make_async_copy(rows[b], acc.at[didx.at[q]], ssem[b]).wait()

    for q in range(6):
        idx_start(q, q)
    for b in (0, 1):
        idx_wait(b, b)
        gather_start(b, b)
    pltpu.sync_copy(zeros_hbm, acc.at[my_rows])
    plsc.subcore_barrier()

    @pl.loop(0, (NCH - 8) // 8)
    def _(j):
        for k in range(8):
            g = 2 + 8 * j + k
            b = k % 2
            q = (2 + k) % 8
            qp = k
            gather_wait(qp, b)
            scatter_start(qp, b)
            idx_start(g + 4, (6 + k) % 8)
            scatter_wait(qp, b)
            idx_wait(g, q)
            gather_start(q, b)

    for k in range(6):
        g = NCH - 6 + k
        b = g % 2
        q = g % 8
        qp = (g - 2) % 8
        gather_wait(qp, b)
        scatter_start(qp, b)
        if g + 4 < NCH:
            idx_start(g + 4, (g + 4) % 8)
        scatter_wait(qp, b)
        idx_wait(g, q)
        gather_start(q, b)
    for k in (0, 1):
        g = NCH - 2 + k
        b = g % 2
        q = g % 8
        gather_wait(q, b)
        scatter_start(q, b)
        scatter_wait(q, b)

    plsc.subcore_barrier()
    pltpu.sync_copy(acc.at[my_rows], part_hbm.at[cid, my_rows])


@functools.partial(
    pl.kernel,
    out_type=jax.ShapeDtypeStruct((N_CORES, DROWS, D), jnp.float32),
    mesh=_mesh,
    scratch_types=[
        pltpu.VMEM_SHARED((DROWS, D), jnp.float32),
        pltpu.VMEM((NCH, CHUNK), jnp.int32),
        pltpu.VMEM((DROWS, D), jnp.float32),
        pltpu.VMEM((DROWS,), jnp.int32),
    ],
    compiler_params=_cp,
)
def _deg(dst_hbm, iota_hbm, zeros_hbm, degp_hbm, acc, didx, hist, iota_v):
    cid = lax.axis_index("c")
    sid = lax.axis_index("s")
    wid = cid * N_SUBCORES + sid
    zv = jnp.zeros((16,), jnp.float32)
    ones_v = jnp.ones((16,), jnp.float32)

    @pl.when(sid < DROWS // 8)
    def _():
        pltpu.sync_copy(zeros_hbm.at[pl.ds(0, 8)],
                        acc.at[pl.ds(sid * 8, 8)])

    pltpu.sync_copy(dst_hbm.at[wid], didx)
    pltpu.sync_copy(iota_hbm, iota_v)

    @pl.loop(0, DROWS)
    def _(r):
        for k in range(D // 16):
            hist[r, pl.ds(k * 16, 16)] = zv

    @pl.loop(0, NCH)
    def _(j):
        for k in range(CHUNK // 16):
            idx16 = didx[j, pl.ds(k * 16, 16)]
            r = lax.shift_right_logical(idx16, 7)
            c = lax.bitwise_and(idx16, 127)
            plsc.addupdate_scatter(hist, [r, c], ones_v)

    plsc.subcore_barrier()
    pltpu.sync_copy(hist, acc.at[iota_v], add=True)
    plsc.subcore_barrier()

    @pl.when(sid < DROWS // 8)
    def _():
        sl = pl.ds(sid * 8, 8)
        pltpu.sync_copy(acc.at[sl], degp_hbm.at[cid, sl])


def _norm_t(degp_ref):
    deg2d = degp_ref[0] + degp_ref[1]
    return lax.rsqrt(jnp.maximum(deg2d, 1.0)).T


def _tc_norm_body(feat_ref, degp_ref, xs_ref):
    nt = _norm_t(degp_ref)
    for r in range(DROWS):
        sl = pl.ds(r * D, D)
        nb = jnp.broadcast_to(nt[:, r:r + 1], (D, D))
        xs_ref[sl, :] = feat_ref[sl, :] * nb


def _tc_mid_body(p_ref, degp_ref, out_ref):
    nt = _norm_t(degp_ref)
    nsq = nt * nt
    for r in range(DROWS):
        sl = pl.ds(r * D, D)
        nb = jnp.broadcast_to(nsq[:, r:r + 1], (D, D))
        out_ref[sl, :] = (p_ref[0, sl, :] + p_ref[1, sl, :]) * nb


def _tc_out_body(p_ref, degp_ref, w_ref, b_ref, out_ref):
    nt = _norm_t(degp_ref)
    w = w_ref[...]
    bb = b_ref[...]
    n_full = N // D
    for r in range(n_full + 1):
        sl = pl.ds(r * D, D)
        nb = jnp.broadcast_to(nt[:, r:r + 1], (D, D))
        t = (p_ref[0, sl, :] + p_ref[1, sl, :]) * nb
        y = lax.dot_general(t, w, (((1,), (0,)), ((), ())),
                            precision=lax.Precision.HIGHEST,
                            preferred_element_type=jnp.float32) + bb
        if r < n_full:
            out_ref[sl, :] = y
        else:
            rem = N - n_full * D
            out_ref[pl.ds(n_full * D, rem), :] = y[:rem]


def kernel(in_feat, edge_index, W, b):
    src = edge_index[0]
    dst = edge_index[1]
    n_pad_e = EPAD - E
    pad_idx = jnp.asarray((np.arange(n_pad_e, dtype=np.int32) % 16) + N)
    src_p = jnp.concatenate([src, pad_idx]).reshape(NW, NCH, CHUNK)
    dst_p = jnp.concatenate([dst, pad_idx]).reshape(NW, NCH, CHUNK)
    feat_p = jnp.pad(in_feat, ((0, NPAD - N), (0, 0)))
    zeros_d = jnp.zeros((RPT, D), jnp.float32)
    iota80 = jnp.arange(DROWS, dtype=jnp.int32)

    degp = _deg(dst_p, iota80, zeros_d)

    xs = pl.pallas_call(
        _tc_norm_body,
        out_shape=jax.ShapeDtypeStruct((NPAD, D), jnp.float32),
    )(feat_p, degp)

    p1 = _hop(xs, src_p, dst_p, zeros_d)

    xs2 = pl.pallas_call(
        _tc_mid_body,
        out_shape=jax.ShapeDtypeStruct((NPAD, D), jnp.float32),
    )(p1, degp)

    p2 = _hop(xs2, src_p, dst_p, zeros_d)

    out = pl.pallas_call(
        _tc_out_body,
        out_shape=jax.ShapeDtypeStruct((N, D), jnp.float32),
    )(p2, degp, W, b)
    return out


# speedup vs baseline: 1.0373x; 1.0373x over previous
import dataclasses
import functools

import jax
import jax.numpy as jnp
import numpy as np
from jax import lax
from jax.experimental import pallas as pl
from jax.experimental.pallas import tpu as pltpu
from jax.experimental.pallas import tpu_sc as plsc

N = 10000
E = 320000
D = 128
N_CORES = 2
N_SUBCORES = 16
NW = N_CORES * N_SUBCORES
CHUNK = 128
NCH = 80
EPAD = NW * NCH * CHUNK
NPAD = 10240
RPT = NPAD // N_SUBCORES
DROWS = NPAD // D

_mesh = plsc.VectorSubcoreMesh(core_axis_name="c", subcore_axis_name="s")

_cp = pltpu.CompilerParams()
if "needs_layout_passes" in pltpu.CompilerParams.__dataclass_fields__:
    _cp = dataclasses.replace(_cp, needs_layout_passes=False)


@functools.partial(
    pl.kernel,
    out_type=jax.ShapeDtypeStruct((N_CORES, NPAD, D), jnp.float32),
    mesh=_mesh,
    scratch_types=[
        pltpu.VMEM_SHARED((NPAD, D), jnp.float32),
        pltpu.VMEM((8, CHUNK), jnp.int32),
        pltpu.VMEM((8, CHUNK), jnp.int32),
        pltpu.VMEM((CHUNK, D), jnp.float32),
        pltpu.VMEM((CHUNK, D), jnp.float32),
        pltpu.SemaphoreType.DMA,
        pltpu.SemaphoreType.DMA,
        pltpu.SemaphoreType.DMA,
        pltpu.SemaphoreType.DMA,
        pltpu.SemaphoreType.DMA,
        pltpu.SemaphoreType.DMA,
        pltpu.SemaphoreType.DMA,
        pltpu.SemaphoreType.DMA,
        pltpu.SemaphoreType.DMA,
        pltpu.SemaphoreType.DMA,
        pltpu.SemaphoreType.DMA,
        pltpu.SemaphoreType.DMA,
    ],
)
def _hop(x_hbm, edge_hbm, zeros_hbm, part_hbm,
         acc, sidx, didx, rows0, rows1,
         isem0, isem1, isem2, isem3, isem4, isem5, isem6, isem7,
         gsem0, gsem1, ssem0, ssem1):
    cid = lax.axis_index("c")
    sid = lax.axis_index("s")
    wid = cid * N_SUBCORES + sid
    my_rows = pl.ds(sid * RPT, RPT)
    isem = (isem0, isem1, isem2, isem3, isem4, isem5, isem6, isem7)
    rows = (rows0, rows1)
    gsem = (gsem0, gsem1)
    ssem = (ssem0, ssem1)

    def idx_start(g, q):
        pltpu.async_copy(edge_hbm.at[0, wid, g], sidx.at[q], isem[q])
        pltpu.async_copy(edge_hbm.at[1, wid, g], didx.at[q], isem[q])

    def idx_wait(g, q):
        pltpu.make_async_copy(edge_hbm.at[0, wid, g], sidx.at[q], isem[q]).wait()
        pltpu.make_async_copy(edge_hbm.at[1, wid, g], didx.at[q], isem[q]).wait()

    def gather_start(q, b):
        pltpu.async_copy(x_hbm.at[sidx.at[q]], rows[b], gsem[b])

    def gather_wait(q, b):
        pltpu.make_async_copy(x_hbm.at[sidx.at[q]], rows[b], gsem[b]).wait()

    def scatter_start(q, b):
        pltpu.async_copy(rows[b], acc.at[didx.at[q]], ssem[b], add=True)

    def scatter_wait(q, b):
        pltpu.make_async_copy(rows[b], acc.at[didx.at[q]], ssem[b]).wait()

    for q in range(6):
        idx_start(q, q)
    for b in (0, 1):
        idx_wait(b, b)
        gather_start(b, b)
    pltpu.sync_copy(zeros_hbm, acc.at[my_rows])
    plsc.subcore_barrier()

    @pl.loop(0, (NCH - 8) // 8)
    def _(j):
        for k in range(8):
            g = 2 + 8 * j + k
            b = k % 2
            q = (2 + k) % 8
            qp = k
            gather_wait(qp, b)
            scatter_start(qp, b)
            idx_start(g + 4, (6 + k) % 8)
            scatter_wait(qp, b)
            idx_wait(g, q)
            gather_start(q, b)

    for k in range(6):
        g = NCH - 6 + k
        b = g % 2
        q = g % 8
        qp = (g - 2) % 8
        gather_wait(qp, b)
        scatter_start(qp, b)
        if g + 4 < NCH:
            idx_start(g + 4, (g + 4) % 8)
        scatter_wait(qp, b)
        idx_wait(g, q)
        gather_start(q, b)
    for k in (0, 1):
        g = NCH - 2 + k
        b = g % 2
        q = g % 8
        gather_wait(q, b)
        scatter_start(q, b)
        scatter_wait(q, b)

    plsc.subcore_barrier()
    pltpu.sync_copy(acc.at[my_rows], part_hbm.at[cid, my_rows])


@functools.partial(
    pl.kernel,
    out_type=jax.ShapeDtypeStruct((N_CORES, DROWS, D), jnp.float32),
    mesh=_mesh,
    scratch_types=[
        pltpu.VMEM_SHARED((DROWS, D), jnp.float32),
        pltpu.VMEM((NCH, CHUNK), jnp.int32),
        pltpu.VMEM((DROWS, D), jnp.float32),
        pltpu.VMEM((DROWS,), jnp.int32),
    ],
    compiler_params=_cp,
)
def _deg(edge_hbm, iota_hbm, zeros_hbm, degp_hbm, acc, didx, hist, iota_v):
    cid = lax.axis_index("c")
    sid = lax.axis_index("s")
    wid = cid * N_SUBCORES + sid
    zv = jnp.zeros((16,), jnp.float32)
    ones_v = jnp.ones((16,), jnp.float32)

    @pl.when(sid < DROWS // 8)
    def _():
        pltpu.sync_copy(zeros_hbm.at[pl.ds(0, 8)],
                        acc.at[pl.ds(sid * 8, 8)])

    pltpu.sync_copy(edge_hbm.at[1, wid], didx)
    pltpu.sync_copy(iota_hbm, iota_v)

    @pl.loop(0, DROWS)
    def _(r):
        for k in range(D // 16):
            hist[r, pl.ds(k * 16, 16)] = zv

    @pl.loop(0, NCH)
    def _(j):
        for k in range(CHUNK // 16):
            idx16 = didx[j, pl.ds(k * 16, 16)]
            r = lax.shift_right_logical(idx16, 7)
            c = lax.bitwise_and(idx16, 127)
            plsc.addupdate_scatter(hist, [r, c], ones_v)

    plsc.subcore_barrier()
    pltpu.sync_copy(hist, acc.at[iota_v], add=True)
    plsc.subcore_barrier()

    @pl.when(sid < DROWS // 8)
    def _():
        sl = pl.ds(sid * 8, 8)
        pltpu.sync_copy(acc.at[sl], degp_hbm.at[cid, sl])


def _norm_t(degp_ref):
    deg2d = degp_ref[0] + degp_ref[1]
    return lax.rsqrt(jnp.maximum(deg2d, 1.0)).T


def _tc_norm_body(feat_ref, degp_ref, xs_ref):
    nt = _norm_t(degp_ref)
    for r in range(DROWS):
        sl = pl.ds(r * D, D)
        nb = jnp.broadcast_to(nt[:, r:r + 1], (D, D))
        xs_ref[sl, :] = feat_ref[sl, :] * nb


def _tc_mid_body(p_ref, degp_ref, out_ref):
    nt = _norm_t(degp_ref)
    nsq = nt * nt
    for r in range(DROWS):
        sl = pl.ds(r * D, D)
        nb = jnp.broadcast_to(nsq[:, r:r + 1], (D, D))
        out_ref[sl, :] = (p_ref[0, sl, :] + p_ref[1, sl, :]) * nb


def _tc_out_body(p_ref, degp_ref, w_ref, b_ref, out_ref):
    nt = _norm_t(degp_ref)
    w = w_ref[...]
    bb = b_ref[...]
    n_full = N // D
    for r in range(n_full + 1):
        sl = pl.ds(r * D, D)
        nb = jnp.broadcast_to(nt[:, r:r + 1], (D, D))
        t = (p_ref[0, sl, :] + p_ref[1, sl, :]) * nb
        y = lax.dot_general(t, w, (((1,), (0,)), ((), ())),
                            precision=lax.Precision.HIGHEST,
                            preferred_element_type=jnp.float32) + bb
        if r < n_full:
            out_ref[sl, :] = y
        else:
            rem = N - n_full * D
            out_ref[pl.ds(n_full * D, rem), :] = y[:rem]


def kernel(in_feat, edge_index, W, b):
    n_pad_e = EPAD - E
    pad_2d = jnp.asarray(
        np.broadcast_to((np.arange(n_pad_e, dtype=np.int32) % 16) + N,
                        (2, n_pad_e)))
    edges_p = jnp.concatenate([edge_index, pad_2d], axis=1)
    edges_p = edges_p.reshape(2, NW, NCH, CHUNK)
    feat_p = jnp.pad(in_feat, ((0, NPAD - N), (0, 0)))
    zeros_d = jnp.zeros((RPT, D), jnp.float32)
    iota80 = jnp.arange(DROWS, dtype=jnp.int32)

    degp = _deg(edges_p, iota80, zeros_d)

    xs = pl.pallas_call(
        _tc_norm_body,
        out_shape=jax.ShapeDtypeStruct((NPAD, D), jnp.float32),
    )(feat_p, degp)

    p1 = _hop(xs, edges_p, zeros_d)

    xs2 = pl.pallas_call(
        _tc_mid_body,
        out_shape=jax.ShapeDtypeStruct((NPAD, D), jnp.float32),
    )(p1, degp)

    p2 = _hop(xs2, edges_p, zeros_d)

    out = pl.pallas_call(
        _tc_out_body,
        out_shape=jax.ShapeDtypeStruct((N, D), jnp.float32),
    )(p2, degp, W, b)
    return out
